# baseline (speedup 1.0000x reference)
import jax
import jax.numpy as jnp
from jax import lax
from jax.experimental import pallas as pl
from jax.experimental.pallas import tpu as pltpu
from jax.experimental.pallas import tpu_sc as plsc

B, N, T, H = 8, 325, 168, 64
C = 2
CH = T * H // C

NW = 32
BT_PAD = 1536
R_PW = BT_PAD // NW


def _sc_gather_body(tab_hbm, idx_hbm, out_hbm, idx_v, rows_v, sem):
    wid = lax.axis_index("s") * 2 + lax.axis_index("c")
    base = wid * R_PW
    pltpu.sync_copy(idx_hbm.at[pl.ds(base, R_PW)], idx_v)
    pltpu.async_copy(tab_hbm.at[idx_v], rows_v, sem).wait()
    pltpu.sync_copy(rows_v, out_hbm.at[pl.ds(base, R_PW)])


def _add_body(node_ref, add_ref, out_ref):
    out_ref[...] = node_ref[...] + add_ref[...]


def kernel(node_emb, hour_of_day, session_start, session_end, position_emb):
    table = (position_emb.at[0].add(session_start)
             .at[23].add(session_end))
    tab2 = jnp.tile(table, (1, 2))
    idx = jnp.pad(hour_of_day.astype(jnp.int32).reshape(B * T),
                  (0, BT_PAD - B * T), constant_values=1)
    mesh = plsc.VectorSubcoreMesh(core_axis_name="c", subcore_axis_name="s")
    sc_gather = pl.kernel(
        _sc_gather_body,
        out_type=jax.ShapeDtypeStruct((BT_PAD, 128), jnp.float32),
        mesh=mesh,
        scratch_types=[
            pltpu.VMEM((R_PW,), jnp.int32),
            pltpu.VMEM((R_PW, 128), jnp.float32),
            pltpu.SemaphoreType.DMA,
        ],
    )
    add = sc_gather(tab2, idx)[:B * T, :H]

    node2 = node_emb.reshape(B, N, T * H)
    add2 = add.reshape(B, 1, T * H)
    out2 = pl.pallas_call(
        _add_body,
        grid=(B, C),
        in_specs=[
            pl.BlockSpec((1, N, CH), lambda b, c: (b, 0, c)),
            pl.BlockSpec((1, 1, CH), lambda b, c: (b, 0, c)),
        ],
        out_specs=pl.BlockSpec((1, N, CH), lambda b, c: (b, 0, c)),
        out_shape=jax.ShapeDtypeStruct((B, N, T * H), jnp.float32),
        compiler_params=pltpu.CompilerParams(
            dimension_semantics=("parallel", "parallel")),
    )(node2, add2)
    return out2.reshape(B, N, T, H)


# speedup vs baseline: 1.0023x; 1.0023x over previous
import jax
import jax.numpy as jnp
from jax import lax
from jax.experimental import pallas as pl
from jax.experimental.pallas import tpu as pltpu
from jax.experimental.pallas import tpu_sc as plsc

B, N, T, H = 8, 325, 168, 64
C = 1
CH = T * H // C

NW = 32
BT_PAD = 1536
R_PW = BT_PAD // NW


def _sc_gather_body(tab_hbm, idx_hbm, out_hbm, idx_v, rows_v, sem):
    wid = lax.axis_index("s") * 2 + lax.axis_index("c")
    base = wid * R_PW
    pltpu.sync_copy(idx_hbm.at[pl.ds(base, R_PW)], idx_v)
    pltpu.async_copy(tab_hbm.at[idx_v], rows_v, sem).wait()
    pltpu.sync_copy(rows_v, out_hbm.at[pl.ds(base, R_PW)])


def _add_body(node_ref, add_ref, out_ref):
    out_ref[...] = node_ref[...] + add_ref[...]


def kernel(node_emb, hour_of_day, session_start, session_end, position_emb):
    table = (position_emb.at[0].add(session_start)
             .at[23].add(session_end))
    tab2 = jnp.tile(table, (1, 2))
    idx = jnp.pad(hour_of_day.astype(jnp.int32).reshape(B * T),
                  (0, BT_PAD - B * T), constant_values=1)
    mesh = plsc.VectorSubcoreMesh(core_axis_name="c", subcore_axis_name="s")
    sc_gather = pl.kernel(
        _sc_gather_body,
        out_type=jax.ShapeDtypeStruct((BT_PAD, 128), jnp.float32),
        mesh=mesh,
        scratch_types=[
            pltpu.VMEM((R_PW,), jnp.int32),
            pltpu.VMEM((R_PW, 128), jnp.float32),
            pltpu.SemaphoreType.DMA,
        ],
    )
    add = sc_gather(tab2, idx)[:B * T, :H]

    node2 = node_emb.reshape(B, N, T * H)
    add2 = add.reshape(B, 1, T * H)
    out2 = pl.pallas_call(
        _add_body,
        grid=(B, C),
        in_specs=[
            pl.BlockSpec((1, N, CH), lambda b, c: (b, 0, c)),
            pl.BlockSpec((1, 1, CH), lambda b, c: (b, 0, c)),
        ],
        out_specs=pl.BlockSpec((1, N, CH), lambda b, c: (b, 0, c)),
        out_shape=jax.ShapeDtypeStruct((B, N, T * H), jnp.float32),
        compiler_params=pltpu.CompilerParams(
            dimension_semantics=("parallel", "parallel")),
    )(node2, add2)
    return out2.reshape(B, N, T, H)
